# Initial kernel scaffold; baseline (speedup 1.0000x reference)
import jax
import jax.numpy as jnp
from jax.experimental import pallas as pl


def kernel(feature, edge_index, W1_l, W1_r, b1, W2_l, W2_r, b2):
    raise NotImplementedError("write your pallas kernel here")


# speedup vs baseline: 8.7754x; 8.7754x over previous
import functools

import jax
import jax.numpy as jnp
from jax import lax
from jax.experimental import pallas as pl
from jax.experimental.pallas import tpu as pltpu
from jax.experimental.pallas import tpu_sc as plsc

_N = 10000
_E = 320000
_DIN = 128
_DH = 256
_DOUT = 128
_W = 128

_NC = 2
_NS = 16
_NW = _NC * _NS
_EPW = _E // _NW
_CH = 80
_NCH = _EPW // _CH
_NST = 5
_CPS = _NCH // _NST
_NP = 10240
_RPT = _NP // _NS
_L = 16


def _make_segsum(with_deg):
    mesh = plsc.VectorSubcoreMesh(core_axis_name="c", subcore_axis_name="s")

    out_type = [jax.ShapeDtypeStruct((_NC, _NP, _W), jnp.float32)]
    scratch = [
        pltpu.VMEM((_CPS, _CH), jnp.int32),
        pltpu.VMEM((_CPS, _CH), jnp.int32),
        pltpu.VMEM((_CH, _W), jnp.float32),
        pltpu.VMEM_SHARED((_NP, _W), jnp.float32),
        pltpu.SemaphoreType.DMA,
    ]
    if with_deg:
        out_type.append(jax.ShapeDtypeStruct((_NC, _NP), jnp.float32))
        scratch += [
            pltpu.VMEM((_NP,), jnp.float32),
            pltpu.VMEM_SHARED((_NS, _NP), jnp.float32),
            pltpu.VMEM((_RPT,), jnp.float32),
        ]

    @functools.partial(
        pl.kernel,
        out_type=tuple(out_type),
        mesh=mesh,
        scratch_types=scratch,
        compiler_params=pltpu.CompilerParams(needs_layout_passes=False),
    )
    def seg(table, src4, dst4, zrows, zdeg, out, *rest):
        if with_deg:
            out_deg, src_v, dst_v, rows_v, acc, sem, deg_v, deg_sh, tmp_v = rest
        else:
            src_v, dst_v, rows_v, acc, sem = rest
        c = lax.axis_index("c")
        s = lax.axis_index("s")
        wid = s * _NC + c

        pltpu.sync_copy(zrows, acc.at[pl.ds(s * _RPT, _RPT)])
        if with_deg:
            pltpu.sync_copy(zdeg, deg_v)
        plsc.subcore_barrier()

        def chunk(g, carry):
            pltpu.async_copy(table.at[src_v.at[g]], rows_v, sem).wait()
            pltpu.sync_copy(rows_v, acc.at[dst_v.at[g]], add=True)
            if with_deg:
                for k in range(_CH // _L):
                    d16 = dst_v[g, pl.ds(k * _L, _L)]
                    cnt, last = plsc.scan_count(d16)
                    plsc.addupdate_scatter(
                        deg_v, [d16], cnt.astype(jnp.float32), mask=last)
            return carry

        def stage(st, carry):
            pltpu.sync_copy(src4.at[wid, st], src_v)
            pltpu.sync_copy(dst4.at[wid, st], dst_v)
            lax.fori_loop(0, _CPS, chunk, 0)
            return carry

        lax.fori_loop(0, _NST, stage, 0)

        if with_deg:
            pltpu.sync_copy(deg_v, deg_sh.at[s])
        plsc.subcore_barrier()

        pltpu.sync_copy(acc.at[pl.ds(s * _RPT, _RPT)],
                        out.at[c, pl.ds(s * _RPT, _RPT)])
        if with_deg:
            for t in range(_NS):
                pltpu.sync_copy(deg_sh.at[t, pl.ds(s * _RPT, _RPT)], tmp_v)
                for k in range(_RPT // _L):
                    piece = tmp_v[pl.ds(k * _L, _L)]
                    if t == 0:
                        deg_v[pl.ds(k * _L, _L)] = piece
                    else:
                        deg_v[pl.ds(k * _L, _L)] += piece
            pltpu.sync_copy(deg_v.at[pl.ds(0, _RPT)],
                            out_deg.at[c, pl.ds(s * _RPT, _RPT)])

    return seg


_segsum_deg = _make_segsum(True)
_segsum = _make_segsum(False)


def _tc_layer1(parts1, degp, x, w1l, w1r, b1, w2l):
    rows = 1000
    grid = _N // rows

    def body(p_ref, d_ref, x_ref, wl_ref, wr_ref, b_ref, w2_ref,
             h_ref, z_ref, di_ref):
        summed = p_ref[0] + p_ref[1]
        deg = d_ref[0] + d_ref[1]
        deginv = 1.0 / jnp.maximum(deg, 1.0)
        mean = summed * deginv
        h = jnp.dot(mean, wl_ref[...], preferred_element_type=jnp.float32)
        h += jnp.dot(x_ref[...], wr_ref[...], preferred_element_type=jnp.float32)
        h += b_ref[...]
        h = jnp.maximum(h, 0.0)
        h_ref[...] = h
        z_ref[...] = jnp.dot(h, w2_ref[...], preferred_element_type=jnp.float32)
        di_ref[...] = deginv

    return pl.pallas_call(
        body,
        grid=(grid,),
        in_specs=[
            pl.BlockSpec((_NC, rows, _W), lambda i: (0, i, 0)),
            pl.BlockSpec((_NC, rows, 1), lambda i: (0, i, 0)),
            pl.BlockSpec((rows, _DIN), lambda i: (i, 0)),
            pl.BlockSpec((_DIN, _DH), lambda i: (0, 0)),
            pl.BlockSpec((_DIN, _DH), lambda i: (0, 0)),
            pl.BlockSpec((1, _DH), lambda i: (0, 0)),
            pl.BlockSpec((_DH, _DOUT), lambda i: (0, 0)),
        ],
        out_specs=[
            pl.BlockSpec((rows, _DH), lambda i: (i, 0)),
            pl.BlockSpec((rows, _DOUT), lambda i: (i, 0)),
            pl.BlockSpec((rows, 1), lambda i: (i, 0)),
        ],
        out_shape=[
            jax.ShapeDtypeStruct((_N, _DH), jnp.float32),
            jax.ShapeDtypeStruct((_N, _DOUT), jnp.float32),
            jax.ShapeDtypeStruct((_N, 1), jnp.float32),
        ],
    )(parts1, degp, x, w1l, w1r, b1, w2l)


def _tc_layer2(parts2, h, deginv, w2r, b2):
    rows = 1000
    grid = _N // rows

    def body(p_ref, h_ref, di_ref, wr_ref, b_ref, o_ref):
        summed = p_ref[0] + p_ref[1]
        t = summed * di_ref[...]
        t += jnp.dot(h_ref[...], wr_ref[...], preferred_element_type=jnp.float32)
        t += b_ref[...]
        m = jnp.max(t, axis=1, keepdims=True)
        e = jnp.exp(t - m)
        lse = jnp.log(jnp.sum(e, axis=1, keepdims=True))
        o_ref[...] = t - m - lse

    return pl.pallas_call(
        body,
        grid=(grid,),
        in_specs=[
            pl.BlockSpec((_NC, rows, _DOUT), lambda i: (0, i, 0)),
            pl.BlockSpec((rows, _DH), lambda i: (i, 0)),
            pl.BlockSpec((rows, 1), lambda i: (i, 0)),
            pl.BlockSpec((_DH, _DOUT), lambda i: (0, 0)),
            pl.BlockSpec((1, _DOUT), lambda i: (0, 0)),
        ],
        out_specs=pl.BlockSpec((rows, _DOUT), lambda i: (i, 0)),
        out_shape=jax.ShapeDtypeStruct((_N, _DOUT), jnp.float32),
    )(parts2, h, deginv, w2r, b2)


def kernel(feature, edge_index, W1_l, W1_r, b1, W2_l, W2_r, b2):
    src4 = edge_index[0].reshape(_NW, _NST, _CPS, _CH)
    dst4 = edge_index[1].reshape(_NW, _NST, _CPS, _CH)

    zrows = jnp.zeros((_RPT, _W), jnp.float32)
    zdeg = jnp.zeros((_NP,), jnp.float32)

    parts1, deg = _segsum_deg(feature, src4, dst4, zrows, zdeg)
    degp = deg.reshape(_NC, _NP, 1)
    h, z, deginv = _tc_layer1(parts1, degp, feature, W1_l, W1_r,
                              b1.reshape(1, _DH), W2_l)
    (parts2,) = _segsum(z, src4, dst4, zrows, zdeg)
    return _tc_layer2(parts2, h, deginv, W2_r, b2.reshape(1, _DOUT))
